# baseline (speedup 1.0000x reference)
import jax
import jax.numpy as jnp
from jax import lax
from jax.experimental import pallas as pl
from jax.experimental.pallas import tpu as pltpu
from jax.experimental.pallas import tpu_sc as plsc

D_MODEL = 64
NUM_CORES = 2
NUM_SUBCORES = 16
NUM_WORKERS = NUM_CORES * NUM_SUBCORES
LANES = 16
NBUF = 2
CHUNK = 128


def _sc_body(x_hbm, ts_hbm, tab_hbm, out_hbm, tab_v, ts_v,
             x0, x1, r0, r1, tsem0, tsem1, xsem0, xsem1, osem0, osem1):
    xs = (x0, x1)
    rs = (r0, r1)
    tsem = (tsem0, tsem1)
    xsem = (xsem0, xsem1)
    osem = (osem0, osem1)
    n_tok = ts_hbm.shape[0]
    tpw = n_tok // NUM_WORKERS
    nchunk = tpw // CHUNK
    wid = lax.axis_index("s") * NUM_CORES + lax.axis_index("c")
    tok0 = wid * tpw

    pltpu.sync_copy(tab_hbm, tab_v)

    def issue(s, b):
        off = tok0 + s * CHUNK
        pltpu.async_copy(ts_hbm.at[pl.ds(off, CHUNK)], ts_v.at[b], tsem[b])
        pltpu.async_copy(x_hbm.at[pl.ds(off * D_MODEL, CHUNK * D_MODEL)],
                         xs[b], xsem[b])

    for b in range(NBUF):
        issue(b, b)

    @pl.loop(0, nchunk, step=NBUF)
    def _(s0):
        for b in range(NBUF):
            s = s0 + b
            pltpu.make_async_copy(
                ts_hbm.at[pl.ds(0, CHUNK)], ts_v.at[b], tsem[b]).wait()
            pltpu.make_async_copy(
                x_hbm.at[pl.ds(0, CHUNK * D_MODEL)], xs[b], xsem[b]).wait()

            @pl.when(s >= NBUF)
            def _():
                pltpu.make_async_copy(
                    rs[b], out_hbm.at[pl.ds(0, CHUNK * D_MODEL)],
                    osem[b]).wait()

            for g in range(CHUNK // LANES):
                base = ts_v[b, pl.ds(g * LANES, LANES)] * D_MODEL
                tok = (lax.iota(jnp.int32, LANES) + g * LANES) * D_MODEL

                @plsc.parallel_loop(0, D_MODEL, unroll=8)
                def _(d):
                    emb = plsc.load_gather(tab_v, [base + d])
                    xv = plsc.load_gather(xs[b], [tok + d])
                    plsc.store_scatter(rs[b], [tok + d], xv + emb)

            pltpu.async_copy(
                rs[b],
                out_hbm.at[pl.ds((tok0 + s * CHUNK) * D_MODEL,
                                 CHUNK * D_MODEL)],
                osem[b])

            @pl.when(s + NBUF < nchunk)
            def _():
                issue(s + NBUF, b)

    for b in range(NBUF):
        pltpu.make_async_copy(
            rs[b], out_hbm.at[pl.ds(0, CHUNK * D_MODEL)], osem[b]).wait()


@jax.jit
def _sc_call(x_flat, ts_flat, tab_flat):
    n_tok = ts_flat.shape[0]
    mesh = plsc.VectorSubcoreMesh(core_axis_name="c", subcore_axis_name="s")
    f = pl.kernel(
        _sc_body,
        mesh=mesh,
        compiler_params=pltpu.CompilerParams(
            use_tc_tiling_on_sc=False, needs_layout_passes=False),
        out_type=jax.ShapeDtypeStruct((n_tok * D_MODEL,), jnp.float32),
        scratch_types=[
            pltpu.VMEM((tab_flat.shape[0],), jnp.float32),
            pltpu.VMEM((NBUF, CHUNK), jnp.int32),
            pltpu.VMEM((CHUNK * D_MODEL,), jnp.float32),
            pltpu.VMEM((CHUNK * D_MODEL,), jnp.float32),
            pltpu.VMEM((CHUNK * D_MODEL,), jnp.float32),
            pltpu.VMEM((CHUNK * D_MODEL,), jnp.float32),
            pltpu.SemaphoreType.DMA,
            pltpu.SemaphoreType.DMA,
            pltpu.SemaphoreType.DMA,
            pltpu.SemaphoreType.DMA,
            pltpu.SemaphoreType.DMA,
            pltpu.SemaphoreType.DMA,
        ],
    )
    return f(x_flat, ts_flat, tab_flat)


def kernel(x, timestamps, time_embedding):
    b, s, d = x.shape
    out_flat = _sc_call(
        x.reshape(-1),
        timestamps.reshape(-1).astype(jnp.int32),
        time_embedding.reshape(-1),
    )
    return out_flat.reshape(b, s, d)


# speedup vs baseline: 2.0778x; 2.0778x over previous
import jax
import jax.numpy as jnp
from jax import lax
from jax.experimental import pallas as pl
from jax.experimental.pallas import tpu as pltpu
from jax.experimental.pallas import tpu_sc as plsc

D_MODEL = 64
NUM_CORES = 2
NUM_SUBCORES = 16
NUM_WORKERS = NUM_CORES * NUM_SUBCORES
LANES = 16
NBUF = 2
CHUNK = 256


def _sc_body(x_hbm, ts_hbm, tab_hbm, out_hbm, ts_v, emb0, emb1,
             x0, x1, r0, r1, tsem0, tsem1, gsem0, gsem1,
             xsem0, xsem1, osem0, osem1):
    embs = (emb0, emb1)
    xs = (x0, x1)
    rs = (r0, r1)
    tsem = (tsem0, tsem1)
    gsem = (gsem0, gsem1)
    xsem = (xsem0, xsem1)
    osem = (osem0, osem1)
    n_tok = ts_hbm.shape[0]
    tpw = n_tok // NUM_WORKERS
    nchunk = tpw // CHUNK
    wid = lax.axis_index("s") * NUM_CORES + lax.axis_index("c")
    tok0 = wid * tpw

    def issue_ts(s, b):
        off = tok0 + s * CHUNK
        pltpu.async_copy(ts_hbm.at[pl.ds(off, CHUNK)], ts_v.at[b], tsem[b])
        pltpu.async_copy(x_hbm.at[pl.ds(off, CHUNK)], xs[b], xsem[b])

    def issue_gather(b):
        pltpu.make_async_copy(
            ts_hbm.at[pl.ds(0, CHUNK)], ts_v.at[b], tsem[b]).wait()
        pltpu.async_copy(
            tab_hbm.at[plsc.Indices(ts_v.at[b])], embs[b], gsem[b])

    for b in range(NBUF):
        issue_ts(b, b)
    issue_gather(0)

    @pl.loop(0, nchunk, step=NBUF)
    def _(s0):
        for b in range(NBUF):
            s = s0 + b
            @pl.when(s + 1 < nchunk)
            def _():
                issue_gather(1 - b)

            pltpu.make_async_copy(
                tab_hbm.at[pl.ds(0, CHUNK)], embs[b], gsem[b]).wait()
            pltpu.make_async_copy(
                x_hbm.at[pl.ds(0, CHUNK)], xs[b], xsem[b]).wait()

            @pl.when(s >= NBUF)
            def _():
                pltpu.make_async_copy(
                    rs[b], out_hbm.at[pl.ds(0, CHUNK)], osem[b]).wait()

            @plsc.parallel_loop(0, CHUNK, unroll=8)
            def _(i):
                for j in range(0, D_MODEL, LANES):
                    sl = pl.ds(j, LANES)
                    rs[b][i, sl] = xs[b][i, sl] + embs[b][i, sl]

            pltpu.async_copy(
                rs[b], out_hbm.at[pl.ds(tok0 + s * CHUNK, CHUNK)], osem[b])

            @pl.when(s + NBUF < nchunk)
            def _():
                issue_ts(s + NBUF, b)

    for b in range(NBUF):
        pltpu.make_async_copy(
            rs[b], out_hbm.at[pl.ds(0, CHUNK)], osem[b]).wait()


@jax.jit
def _sc_call(x_flat, ts_flat, tab):
    n_tok = ts_flat.shape[0]
    mesh = plsc.VectorSubcoreMesh(core_axis_name="c", subcore_axis_name="s")
    f = pl.kernel(
        _sc_body,
        mesh=mesh,
        compiler_params=pltpu.CompilerParams(
            use_tc_tiling_on_sc=False, needs_layout_passes=False),
        out_type=jax.ShapeDtypeStruct((n_tok, D_MODEL), jnp.float32),
        scratch_types=[
            pltpu.VMEM((NBUF, CHUNK), jnp.int32),
            pltpu.VMEM((CHUNK, D_MODEL), jnp.float32),
            pltpu.VMEM((CHUNK, D_MODEL), jnp.float32),
            pltpu.VMEM((CHUNK, D_MODEL), jnp.float32),
            pltpu.VMEM((CHUNK, D_MODEL), jnp.float32),
            pltpu.VMEM((CHUNK, D_MODEL), jnp.float32),
            pltpu.VMEM((CHUNK, D_MODEL), jnp.float32),
            pltpu.SemaphoreType.DMA,
            pltpu.SemaphoreType.DMA,
            pltpu.SemaphoreType.DMA,
            pltpu.SemaphoreType.DMA,
            pltpu.SemaphoreType.DMA,
            pltpu.SemaphoreType.DMA,
            pltpu.SemaphoreType.DMA,
            pltpu.SemaphoreType.DMA,
        ],
    )
    return f(x_flat, ts_flat, tab)


def kernel(x, timestamps, time_embedding):
    b, s, d = x.shape
    out = _sc_call(
        x.reshape(-1, d),
        timestamps.reshape(-1).astype(jnp.int32),
        time_embedding,
    )
    return out.reshape(b, s, d)
